# Initial kernel scaffold; baseline (speedup 1.0000x reference)
import jax
import jax.numpy as jnp
from jax.experimental import pallas as pl


def kernel(input0, input1):
    raise NotImplementedError("write your pallas kernel here")


# speedup vs baseline: 1.6830x; 1.6830x over previous
import functools

import jax
import jax.numpy as jnp
from jax import lax
from jax.experimental import pallas as pl
from jax.experimental.pallas import tpu as pltpu
from jax.experimental.pallas import tpu_sc as plsc

D = 64
B = 16384 * 50
NW = 32
BPW = B // NW
CHUNK = 128
NCHUNK = BPW // CHUNK

_mesh = plsc.VectorSubcoreMesh(core_axis_name="c", subcore_axis_name="s")


@functools.partial(
    pl.kernel,
    mesh=_mesh,
    out_type=jax.ShapeDtypeStruct((B, D), jnp.float32),
    scratch_types=[
        pltpu.VMEM((NCHUNK, CHUNK), jnp.int32),
        pltpu.VMEM((2, CHUNK, D), jnp.float32),
        pltpu.SemaphoreType.DMA,
    ],
    compiler_params=pltpu.CompilerParams(use_tc_tiling_on_sc=False),
)
def _gather_sc(table_hbm, idx_hbm, out_hbm, idx_v, rows_v, gsem):
    wid = lax.axis_index("s") * 2 + lax.axis_index("c")
    row0 = wid * NCHUNK

    pltpu.sync_copy(idx_hbm.at[pl.ds(row0, NCHUNK)], idx_v)

    def body(c, _):
        pltpu.async_copy(table_hbm.at[idx_v.at[c]], rows_v.at[0], gsem).wait()
        pltpu.sync_copy(
            rows_v.at[0], out_hbm.at[pl.ds((row0 + c) * CHUNK, CHUNK)]
        )
        return 0

    lax.fori_loop(0, NCHUNK, body, 0)


def kernel(input0, input1):
    idx = input1.astype(jnp.int32).reshape(B // CHUNK, CHUNK)
    out = _gather_sc(input0, idx)
    return out.reshape(input1.shape + (D,))


# speedup vs baseline: 1.8299x; 1.0873x over previous
import functools, jax, jax.numpy as jnp
from jax import lax
from jax.experimental import pallas as pl
from jax.experimental.pallas import tpu as pltpu
from jax.experimental.pallas import tpu_sc as plsc

D=64; B=16384*50; NW=32; BPW=B//NW; CHUNK=512; NCHUNK=BPW//CHUNK
_mesh = plsc.VectorSubcoreMesh(core_axis_name="c", subcore_axis_name="s")

@functools.partial(pl.kernel, mesh=_mesh,
    out_type=jax.ShapeDtypeStruct((B, D), jnp.float32),
    scratch_types=[pltpu.VMEM((BPW,), jnp.int32),
                   pltpu.VMEM((CHUNK, D), jnp.float32),
                   pltpu.SemaphoreType.DMA],
    compiler_params=pltpu.CompilerParams(use_tc_tiling_on_sc=False))
def _g(table_hbm, idx_hbm, out_hbm, idx_v, rows_v, gsem):
    wid = lax.axis_index("s") * 2 + lax.axis_index("c")
    base = wid * BPW
    pltpu.sync_copy(idx_hbm.at[pl.ds(base, BPW)], idx_v)
    def body(c, _):
        pltpu.async_copy(table_hbm.at[idx_v.at[pl.ds(c*CHUNK, CHUNK)]], rows_v, gsem).wait()
        pltpu.sync_copy(rows_v, out_hbm.at[pl.ds(base + c*CHUNK, CHUNK)])
        return 0
    lax.fori_loop(0, NCHUNK, body, 0)


def kernel(input0, input1):
    idx = input1.astype(jnp.int32).reshape(B)
    out = _g(input0, idx)
    return out.reshape(input1.shape + (D,))


# speedup vs baseline: 1.8763x; 1.0253x over previous
import functools

import jax
import jax.numpy as jnp
from jax import lax
from jax.experimental import pallas as pl
from jax.experimental.pallas import tpu as pltpu
from jax.experimental.pallas import tpu_sc as plsc

D = 64
B = 16384 * 50
NW = 32
BPW = B // NW
CHUNK = 256
NCHUNK = BPW // CHUNK
NBUF = 4
NROUNDS = NCHUNK // NBUF

_mesh = plsc.VectorSubcoreMesh(core_axis_name="c", subcore_axis_name="s")


@functools.partial(
    pl.kernel,
    mesh=_mesh,
    out_type=jax.ShapeDtypeStruct((B, D), jnp.float32),
    scratch_types=[
        pltpu.VMEM((BPW,), jnp.int32),
        pltpu.VMEM((NBUF, CHUNK, D), jnp.float32),
        pltpu.SemaphoreType.DMA,
        pltpu.SemaphoreType.DMA,
    ],
    compiler_params=pltpu.CompilerParams(use_tc_tiling_on_sc=False),
)
def _gather_sc(table_hbm, idx_hbm, out_hbm, idx_v, rows_v, gsem, ssem):
    wid = lax.axis_index("s") * 2 + lax.axis_index("c")
    base = wid * BPW

    pltpu.sync_copy(idx_hbm.at[pl.ds(base, BPW)], idx_v)

    def g_copy(c, b):
        return pltpu.make_async_copy(
            table_hbm.at[idx_v.at[pl.ds(c * CHUNK, CHUNK)]], rows_v.at[b], gsem
        )

    def s_copy(c, b):
        return pltpu.make_async_copy(
            rows_v.at[b], out_hbm.at[pl.ds(base + c * CHUNK, CHUNK)], ssem
        )

    for b in range(NBUF):
        g_copy(b, b).start()

    def body(r, _):
        for b in range(NBUF):
            c = r * NBUF + b
            g_copy(c, b).wait()
            s_copy(c, b).start()
            s_copy(c, b).wait()
            g_copy(c + NBUF, b).start()
        return 0

    lax.fori_loop(0, NROUNDS - 1, body, 0)

    last = (NROUNDS - 1) * NBUF
    for b in range(NBUF):
        g_copy(last + b, b).wait()
        s_copy(last + b, b).start()
    for b in range(NBUF):
        s_copy(last + b, b).wait()


def kernel(input0, input1):
    idx = input1.astype(jnp.int32).reshape(B)
    out = _gather_sc(input0, idx)
    return out.reshape(input1.shape + (D,))
